# baseline (speedup 1.0000x reference)
import jax
import jax.numpy as jnp
from jax import lax
from jax.experimental import pallas as pl

_MULT_SIZE = 1.2


def _pad_body(species_ref, natoms_ref, batch_ref, coords_ref, cells_ref,
              species_out_ref, natoms_out_ref, batch_out_ref, coords_out_ref,
              cells_out_ref, true_atoms_ref, true_sys_ref):
    nat = species_ref.shape[0]
    nsys = natoms_ref.shape[0]
    pad_nat = species_out_ref.shape[0]
    add = pad_nat - nat

    s = species_ref[...]
    species_out_ref[0:nat] = s
    species_out_ref[nat:pad_nat] = jnp.full((add,), -1, species_ref.dtype)
    true_atoms_ref[0:nat] = s > 0
    true_atoms_ref[nat:pad_nat] = jnp.zeros((add,), jnp.bool_)

    batch_out_ref[0:nat] = batch_ref[...]
    batch_out_ref[nat:pad_nat] = jnp.full((add,), nsys, batch_ref.dtype)

    coords_out_ref[0:nat, :] = coords_ref[...]
    coords_out_ref[nat:pad_nat, :] = jnp.zeros((add,) + coords_ref.shape[1:],
                                               coords_ref.dtype)

    natoms_out_ref[0:nsys] = natoms_ref[...]
    natoms_out_ref[nsys:nsys + 1] = jnp.full((1,), add, natoms_ref.dtype)

    cells_out_ref[0:nsys] = cells_ref[...]
    i = lax.broadcasted_iota(jnp.int32, (1, 3, 3), 1)
    j = lax.broadcasted_iota(jnp.int32, (1, 3, 3), 2)
    cells_out_ref[nsys:nsys + 1] = (i == j).astype(cells_ref.dtype)

    true_sys_ref[0:nsys] = jnp.ones((nsys,), jnp.bool_)
    true_sys_ref[nsys:nsys + 1] = jnp.zeros((1,), jnp.bool_)


def kernel(species, natoms, batch_index, coordinates, cells):
    nat = species.shape[0]
    nsys = natoms.shape[0]
    pad_nat = int(_MULT_SIZE * nat) + 1

    out_shape = (
        jax.ShapeDtypeStruct((pad_nat,), species.dtype),
        jax.ShapeDtypeStruct((nsys + 1,), natoms.dtype),
        jax.ShapeDtypeStruct((pad_nat,), batch_index.dtype),
        jax.ShapeDtypeStruct((pad_nat,) + coordinates.shape[1:], coordinates.dtype),
        jax.ShapeDtypeStruct((nsys + 1,) + cells.shape[1:], cells.dtype),
        jax.ShapeDtypeStruct((pad_nat,), jnp.bool_),
        jax.ShapeDtypeStruct((nsys + 1,), jnp.bool_),
    )
    return pl.pallas_call(_pad_body, out_shape=out_shape)(
        species, natoms, batch_index, coordinates, cells)
